# baseline (speedup 1.0000x reference)
import functools
import math

import numpy as np
import jax
import jax.numpy as jnp
from jax import lax
from jax.experimental import pallas as pl
from jax.experimental.pallas import tpu as pltpu
from jax.experimental.pallas import tpu_sc as plsc

D = 128
B = 1024
L = 200
N = B * L
SQRTD = math.sqrt(D)
EPS = 1e-6

NC = 2
NS = 16
NW = NC * NS
RW = N // NW
W = 128
NCHUNK = RW // W


def _make_pe() -> np.ndarray:
    position = np.arange(L)[:, None].astype(np.float32)
    div_term = np.exp(
        np.arange(0, D, 2).astype(np.float32) * -(math.log(10000.0) / D))
    pe = np.zeros((L, D), dtype=np.float32)
    pe[:, 0::2] = np.sin(position * div_term)
    pe[:, 1::2] = np.cos(position * div_term)
    return pe[None]


_PE = _make_pe()


_sc_mesh = plsc.VectorSubcoreMesh(core_axis_name="c", subcore_axis_name="s")


@functools.partial(
    pl.kernel,
    mesh=_sc_mesh,
    out_type=jax.ShapeDtypeStruct((N, D), jnp.float32),
    scratch_types=[
        pltpu.VMEM((RW,), jnp.int32),
        pltpu.VMEM((W, D), jnp.float32),
        pltpu.VMEM((W, D), jnp.float32),
        pltpu.SemaphoreType.DMA,
        pltpu.SemaphoreType.DMA,
    ],
)
def _sc_gather(lut_hbm, idx_hbm, out_hbm, idx_v, buf0, buf1, sem0, sem1):
    wid = lax.axis_index("s") * NC + lax.axis_index("c")
    base = wid * RW
    pltpu.sync_copy(idx_hbm.at[pl.ds(base, RW)], idx_v)

    bufs = (buf0, buf1)
    sems = (sem0, sem1)

    pltpu.async_copy(lut_hbm.at[idx_v.at[pl.ds(0, W)]], buf0, sem0)

    @pl.loop(0, NCHUNK, step=2)
    def _(k):
        for b in range(2):
            cur = k + b

            @pl.when(cur < NCHUNK)
            def _():
                nxt = cur + 1

                @pl.when(nxt < NCHUNK)
                def _():
                    pltpu.async_copy(
                        lut_hbm.at[idx_v.at[pl.ds(nxt * W, W)]],
                        bufs[(b + 1) % 2], sems[(b + 1) % 2])

                pltpu.make_async_copy(
                    lut_hbm.at[idx_v.at[pl.ds(cur * W, W)]],
                    bufs[b], sems[b]).wait()
                pltpu.sync_copy(bufs[b], out_hbm.at[pl.ds(base + cur * W, W)])


def _ln_body(g_ref, x_ref, pe_ref, a_ref, b_ref, o_ref):
    t = g_ref[...] * SQRTD + pe_ref[...] + x_ref[...]
    mean = jnp.mean(t, axis=-1, keepdims=True)
    c = t - mean
    var = jnp.sum(c * c, axis=-1, keepdims=True) * (1.0 / (D - 1))
    std = jnp.sqrt(var)
    o_ref[...] = a_ref[...] * (c / (std + EPS)) + b_ref[...]


BB = 4


def _tc_layernorm(g3, x, pe, a2, b2):
    grid = (B // BB,)
    return pl.pallas_call(
        _ln_body,
        grid=grid,
        in_specs=[
            pl.BlockSpec((BB, L, D), lambda i: (i, 0, 0)),
            pl.BlockSpec((BB, L, D), lambda i: (i, 0, 0)),
            pl.BlockSpec((1, L, D), lambda i: (0, 0, 0)),
            pl.BlockSpec((1, 1, D), lambda i: (0, 0, 0)),
            pl.BlockSpec((1, 1, D), lambda i: (0, 0, 0)),
        ],
        out_specs=pl.BlockSpec((BB, L, D), lambda i: (i, 0, 0)),
        out_shape=jax.ShapeDtypeStruct((B, L, D), jnp.float32),
    )(g3, x, pe, a2, b2)


def kernel(decodemask, x, lut, a_2, b_2):
    idx = decodemask.reshape(-1).astype(jnp.int32)
    g = _sc_gather(lut, idx)
    g3 = g.reshape(B, L, D)
    pe = jnp.asarray(_PE)
    a2 = a_2.reshape(1, 1, D)
    b2 = b_2.reshape(1, 1, D)
    return _tc_layernorm(g3, x, pe, a2, b2)


# speedup vs baseline: 1.4319x; 1.4319x over previous
import functools
import math

import numpy as np
import jax
import jax.numpy as jnp
from jax import lax
from jax.experimental import pallas as pl
from jax.experimental.pallas import tpu as pltpu
from jax.experimental.pallas import tpu_sc as plsc

D = 128
B = 1024
L = 200
N = B * L
SQRTD = math.sqrt(D)
EPS = 1e-6

NC = 2
NS = 16
NW = NC * NS
RW = N // NW
W = 128
NCHUNK = RW // W


def _make_pe() -> np.ndarray:
    position = np.arange(L)[:, None].astype(np.float32)
    div_term = np.exp(
        np.arange(0, D, 2).astype(np.float32) * -(math.log(10000.0) / D))
    pe = np.zeros((L, D), dtype=np.float32)
    pe[:, 0::2] = np.sin(position * div_term)
    pe[:, 1::2] = np.cos(position * div_term)
    return pe[None]


_PE = _make_pe()


_sc_mesh = plsc.VectorSubcoreMesh(core_axis_name="c", subcore_axis_name="s")


@functools.partial(
    pl.kernel,
    mesh=_sc_mesh,
    out_type=jax.ShapeDtypeStruct((N, D), jnp.float32),
    scratch_types=[
        pltpu.VMEM((RW,), jnp.int32),
        pltpu.VMEM((W, D), jnp.float32),
        pltpu.VMEM((W, D), jnp.float32),
        pltpu.SemaphoreType.DMA,
        pltpu.SemaphoreType.DMA,
    ],
)
def _sc_gather(lut_hbm, idx_hbm, out_hbm, idx_v, buf0, buf1, sem0, sem1):
    wid = lax.axis_index("s") * NC + lax.axis_index("c")
    base = wid * RW
    pltpu.sync_copy(idx_hbm.at[pl.ds(base, RW)], idx_v)

    bufs = (buf0, buf1)
    sems = (sem0, sem1)

    pltpu.async_copy(lut_hbm.at[idx_v.at[pl.ds(0, W)]], buf0, sem0)

    @pl.loop(0, NCHUNK, step=2)
    def _(k):
        for b in range(2):
            cur = k + b

            @pl.when(cur < NCHUNK)
            def _():
                nxt = cur + 1

                @pl.when(nxt < NCHUNK)
                def _():
                    pltpu.async_copy(
                        lut_hbm.at[idx_v.at[pl.ds(nxt * W, W)]],
                        bufs[(b + 1) % 2], sems[(b + 1) % 2])

                pltpu.make_async_copy(
                    lut_hbm.at[idx_v.at[pl.ds(cur * W, W)]],
                    bufs[b], sems[b]).wait()
                pltpu.sync_copy(bufs[b], out_hbm.at[pl.ds(base + cur * W, W)])


def _ln_body(g_ref, x_ref, pe_ref, a_ref, b_ref, o_ref):
    t = g_ref[...] * SQRTD + pe_ref[...] + x_ref[...]
    mean = jnp.mean(t, axis=-1, keepdims=True)
    c = t - mean
    var = jnp.sum(c * c, axis=-1, keepdims=True) * (1.0 / (D - 1))
    std = jnp.sqrt(var)
    o_ref[...] = a_ref[...] * (c / (std + EPS)) + b_ref[...]


BB = 16


def _tc_layernorm(g3, x, pe, a2, b2):
    grid = (B // BB,)
    return pl.pallas_call(
        _ln_body,
        grid=grid,
        in_specs=[
            pl.BlockSpec((BB, L, D), lambda i: (i, 0, 0)),
            pl.BlockSpec((BB, L, D), lambda i: (i, 0, 0)),
            pl.BlockSpec((1, L, D), lambda i: (0, 0, 0)),
            pl.BlockSpec((1, 1, D), lambda i: (0, 0, 0)),
            pl.BlockSpec((1, 1, D), lambda i: (0, 0, 0)),
        ],
        out_specs=pl.BlockSpec((BB, L, D), lambda i: (i, 0, 0)),
        out_shape=jax.ShapeDtypeStruct((B, L, D), jnp.float32),
    )(g3, x, pe, a2, b2)


def kernel(decodemask, x, lut, a_2, b_2):
    idx = decodemask.reshape(-1).astype(jnp.int32)
    g = _sc_gather(lut, idx)
    g3 = g.reshape(B, L, D)
    pe = jnp.asarray(_PE)
    a2 = a_2.reshape(1, 1, D)
    b2 = b_2.reshape(1, 1, D)
    return _tc_layernorm(g3, x, pe, a2, b2)


# speedup vs baseline: 1.5576x; 1.0878x over previous
import functools
import math

import numpy as np
import jax
import jax.numpy as jnp
from jax import lax
from jax.experimental import pallas as pl
from jax.experimental.pallas import tpu as pltpu
from jax.experimental.pallas import tpu_sc as plsc

D = 128
B = 1024
L = 200
N = B * L
SQRTD = math.sqrt(D)
EPS = 1e-6

NC = 2
NS = 16
NW = NC * NS
RW = N // NW
W = 128
NCHUNK = RW // W


def _make_pe() -> np.ndarray:
    position = np.arange(L)[:, None].astype(np.float32)
    div_term = np.exp(
        np.arange(0, D, 2).astype(np.float32) * -(math.log(10000.0) / D))
    pe = np.zeros((L, D), dtype=np.float32)
    pe[:, 0::2] = np.sin(position * div_term)
    pe[:, 1::2] = np.cos(position * div_term)
    return pe[None]


_PE = _make_pe()


_sc_mesh = plsc.VectorSubcoreMesh(core_axis_name="c", subcore_axis_name="s")


@functools.partial(
    pl.kernel,
    mesh=_sc_mesh,
    out_type=jax.ShapeDtypeStruct((N, D), jnp.float32),
    scratch_types=[
        pltpu.VMEM((RW,), jnp.int32),
        pltpu.VMEM((W, D), jnp.float32),
        pltpu.VMEM((W, D), jnp.float32),
        pltpu.SemaphoreType.DMA,
        pltpu.SemaphoreType.DMA,
    ],
)
def _sc_gather(lut_hbm, idx_hbm, out_hbm, idx_v, buf0, buf1, sem0, sem1):
    wid = lax.axis_index("s") * NC + lax.axis_index("c")
    base = wid * RW
    pltpu.sync_copy(idx_hbm.at[pl.ds(base, RW)], idx_v)

    bufs = (buf0, buf1)
    sems = (sem0, sem1)

    pltpu.async_copy(lut_hbm.at[idx_v.at[pl.ds(0, W)]], buf0, sem0)

    @pl.loop(0, NCHUNK, step=2)
    def _(k):
        for b in range(2):
            cur = k + b

            @pl.when(cur < NCHUNK)
            def _():
                nxt = cur + 1

                @pl.when(nxt < NCHUNK)
                def _():
                    pltpu.async_copy(
                        lut_hbm.at[idx_v.at[pl.ds(nxt * W, W)]],
                        bufs[(b + 1) % 2], sems[(b + 1) % 2])

                pltpu.make_async_copy(
                    lut_hbm.at[idx_v.at[pl.ds(cur * W, W)]],
                    bufs[b], sems[b]).wait()
                pltpu.sync_copy(bufs[b], out_hbm.at[pl.ds(base + cur * W, W)])


def _ln_body(g_ref, x_ref, pe_ref, a_ref, b_ref, o_ref):
    t = g_ref[...] * SQRTD + pe_ref[...] + x_ref[...]
    mean = jnp.mean(t, axis=-1, keepdims=True)
    c = t - mean
    var = jnp.sum(c * c, axis=-1, keepdims=True) * (1.0 / (D - 1))
    std = jnp.sqrt(var)
    o_ref[...] = a_ref[...] * (c / (std + EPS)) + b_ref[...]


BB = 32


def _tc_layernorm(g3, x, pe, a2, b2):
    grid = (B // BB,)
    return pl.pallas_call(
        _ln_body,
        grid=grid,
        in_specs=[
            pl.BlockSpec((BB, L, D), lambda i: (i, 0, 0)),
            pl.BlockSpec((BB, L, D), lambda i: (i, 0, 0)),
            pl.BlockSpec((1, L, D), lambda i: (0, 0, 0)),
            pl.BlockSpec((1, 1, D), lambda i: (0, 0, 0)),
            pl.BlockSpec((1, 1, D), lambda i: (0, 0, 0)),
        ],
        out_specs=pl.BlockSpec((BB, L, D), lambda i: (i, 0, 0)),
        out_shape=jax.ShapeDtypeStruct((B, L, D), jnp.float32),
    )(g3, x, pe, a2, b2)


def kernel(decodemask, x, lut, a_2, b_2):
    idx = decodemask.reshape(-1).astype(jnp.int32)
    g = _sc_gather(lut, idx)
    g3 = g.reshape(B, L, D)
    pe = jnp.asarray(_PE)
    a2 = a_2.reshape(1, 1, D)
    b2 = b_2.reshape(1, 1, D)
    return _tc_layernorm(g3, x, pe, a2, b2)


# speedup vs baseline: 1.5860x; 1.0182x over previous
import functools
import math

import numpy as np
import jax
import jax.numpy as jnp
from jax import lax
from jax.experimental import pallas as pl
from jax.experimental.pallas import tpu as pltpu
from jax.experimental.pallas import tpu_sc as plsc

D = 128
B = 1024
L = 200
N = B * L
SQRTD = math.sqrt(D)
EPS = 1e-6

NC = 2
NS = 16
NW = NC * NS
RW = N // NW
W = 128
NCHUNK = RW // W


def _make_pe() -> np.ndarray:
    position = np.arange(L)[:, None].astype(np.float32)
    div_term = np.exp(
        np.arange(0, D, 2).astype(np.float32) * -(math.log(10000.0) / D))
    pe = np.zeros((L, D), dtype=np.float32)
    pe[:, 0::2] = np.sin(position * div_term)
    pe[:, 1::2] = np.cos(position * div_term)
    return pe[None]


_PE = _make_pe()


_sc_mesh = plsc.VectorSubcoreMesh(core_axis_name="c", subcore_axis_name="s")


@functools.partial(
    pl.kernel,
    mesh=_sc_mesh,
    out_type=jax.ShapeDtypeStruct((N, D), jnp.float32),
    scratch_types=[
        pltpu.VMEM((RW,), jnp.int32),
        pltpu.VMEM((W, D), jnp.float32),
        pltpu.VMEM((W, D), jnp.float32),
        pltpu.SemaphoreType.DMA,
        pltpu.SemaphoreType.DMA,
    ],
)
def _sc_gather(lut_hbm, idx_hbm, out_hbm, idx_v, buf0, buf1, sem0, sem1):
    wid = lax.axis_index("s") * NC + lax.axis_index("c")
    base = wid * RW
    pltpu.sync_copy(idx_hbm.at[pl.ds(base, RW)], idx_v)

    bufs = (buf0, buf1)
    sems = (sem0, sem1)

    pltpu.async_copy(lut_hbm.at[idx_v.at[pl.ds(0, W)]], buf0, sem0)

    @pl.loop(0, NCHUNK, step=2)
    def _(k):
        for b in range(2):
            cur = k + b

            @pl.when(cur < NCHUNK)
            def _():
                nxt = cur + 1

                @pl.when(nxt < NCHUNK)
                def _():
                    pltpu.async_copy(
                        lut_hbm.at[idx_v.at[pl.ds(nxt * W, W)]],
                        bufs[(b + 1) % 2], sems[(b + 1) % 2])

                pltpu.make_async_copy(
                    lut_hbm.at[idx_v.at[pl.ds(cur * W, W)]],
                    bufs[b], sems[b]).wait()
                pltpu.sync_copy(bufs[b], out_hbm.at[pl.ds(base + cur * W, W)])


def _ln_body(g_ref, x_ref, pe_ref, a_ref, b_ref, o_ref):
    t = g_ref[...] * SQRTD + pe_ref[...] + x_ref[...]
    mean = jnp.mean(t, axis=-1, keepdims=True)
    c = t - mean
    var = jnp.sum(c * c, axis=-1, keepdims=True) * (1.0 / (D - 1))
    std = jnp.sqrt(var)
    o_ref[...] = a_ref[...] * (c / (std + EPS)) + b_ref[...]


BB = 64


def _tc_layernorm(g3, x, pe, a2, b2):
    grid = (B // BB,)
    return pl.pallas_call(
        _ln_body,
        grid=grid,
        in_specs=[
            pl.BlockSpec((BB, L, D), lambda i: (i, 0, 0)),
            pl.BlockSpec((BB, L, D), lambda i: (i, 0, 0)),
            pl.BlockSpec((1, L, D), lambda i: (0, 0, 0)),
            pl.BlockSpec((1, 1, D), lambda i: (0, 0, 0)),
            pl.BlockSpec((1, 1, D), lambda i: (0, 0, 0)),
        ],
        out_specs=pl.BlockSpec((BB, L, D), lambda i: (i, 0, 0)),
        out_shape=jax.ShapeDtypeStruct((B, L, D), jnp.float32),
    )(g3, x, pe, a2, b2)


def kernel(decodemask, x, lut, a_2, b_2):
    idx = decodemask.reshape(-1).astype(jnp.int32)
    g = _sc_gather(lut, idx)
    g3 = g.reshape(B, L, D)
    pe = jnp.asarray(_PE)
    a2 = a_2.reshape(1, 1, D)
    b2 = b_2.reshape(1, 1, D)
    return _tc_layernorm(g3, x, pe, a2, b2)
